# baseline (speedup 1.0000x reference)
import functools

import jax
import jax.numpy as jnp
from jax import lax
from jax.experimental import pallas as pl
from jax.experimental.pallas import tpu as pltpu, tpu_sc as plsc

NELEM = 1_000_000
NUM_IDS = 16_384
NUM_REGIONS_PAD = 64
NC, NS, L = 2, 16, 16
NW = NC * NS
NSUB = 4
SUB = 7_816
CHUNK = NSUB * SUB
SUB_LAST = NELEM - 31 * CHUNK - 3 * SUB
IDS_PER_TILE = NUM_IDS // NS
NKV = NUM_REGIONS_PAD // L

_MESH = plsc.VectorSubcoreMesh(core_axis_name="c", subcore_axis_name="s")


@functools.partial(
    pl.kernel,
    out_type=jax.ShapeDtypeStruct((NELEM,), jnp.float32),
    mesh=_MESH,
    scratch_types=[
        pltpu.VMEM((CHUNK,), jnp.float32),
        pltpu.VMEM((IDS_PER_TILE,), jnp.int32),
        pltpu.VMEM((NUM_REGIONS_PAD,), jnp.int32),
        pltpu.VMEM((NS, NUM_REGIONS_PAD), jnp.int32),
        pltpu.VMEM((NUM_REGIONS_PAD,), jnp.int32),
        pltpu.VMEM((L,), jnp.float32),
        pltpu.VMEM_SHARED((NS, NUM_REGIONS_PAD), jnp.int32),
        pltpu.SemaphoreType.DMA,
        pltpu.SemaphoreType.DMA,
        pltpu.SemaphoreType.DMA,
    ],
    compiler_params=pltpu.CompilerParams(needs_layout_passes=False),
)
def _region_set_kernel(ids_hbm, eir_hbm, val_hbm, prop_hbm, out_hbm,
                       vbuf, ids_v, hits_v, allhits_v, eir_v, val_v,
                       sh_hits, in_sem, out_sem, cfg_sem):
    c = lax.axis_index("c")
    s = lax.axis_index("s")
    wid = s * NC + c
    base = pl.multiple_of(wid * CHUNK, 8)
    last = wid == NW - 1

    def in_copy(j, sub=SUB):
        return pltpu.make_async_copy(
            prop_hbm.at[pl.ds(base + j * SUB, sub)],
            vbuf.at[pl.ds(j * SUB, sub)], in_sem)

    def out_copy(j, sub=SUB):
        return pltpu.make_async_copy(
            vbuf.at[pl.ds(j * SUB, sub)],
            out_hbm.at[pl.ds(base + j * SUB, sub)], out_sem)

    def branch_last(fn):
        @pl.when(jnp.logical_not(last))
        def _():
            fn(SUB)

        @pl.when(last)
        def _():
            fn(SUB_LAST)

    ids_cp = pltpu.make_async_copy(
        ids_hbm.at[pl.ds(s * IDS_PER_TILE, IDS_PER_TILE)], ids_v, cfg_sem)
    eir_cp = pltpu.make_async_copy(eir_hbm, eir_v, cfg_sem)
    val_cp = pltpu.make_async_copy(val_hbm, val_v, cfg_sem)
    ids_cp.start()
    eir_cp.start()
    val_cp.start()

    for j in range(NSUB - 1):
        in_copy(j).start()
    branch_last(lambda sub: in_copy(NSUB - 1, sub).start())

    ids_cp.wait()
    eir_cp.wait()
    val_cp.wait()

    ts = [eir_v[pl.ds(k * L, L)] for k in range(NKV)]
    chunk_end = base + jnp.where(last, 3 * SUB + SUB_LAST, CHUNK)

    def sub_bounds(j):
        sb = base + j * SUB
        se = jnp.minimum(sb + SUB, chunk_end)
        return sb, se

    def potential(j):
        sb, se = sub_bounds(j)
        acc = jnp.zeros((L,), jnp.int32)
        for k in range(NKV):
            acc = acc + jnp.where((ts[k] >= sb) & (ts[k] < se), 1, 0)
        return lax.reduce_sum_p.bind(acc, axes=(0,)) > 0

    zero16 = jnp.zeros((L,), jnp.int32)
    one16 = jnp.full((L,), 1, jnp.int32)
    for k in range(NKV):
        hits_v[pl.ds(k * L, L)] = zero16
    for k in range(IDS_PER_TILE // L):
        idv = ids_v[pl.ds(k * L, L)]
        plsc.store_scatter(hits_v, [idv], one16)
    pltpu.sync_copy(hits_v, sh_hits.at[s])

    pots = [potential(j) for j in range(NSUB)]
    for j in range(NSUB - 1):
        in_copy(j).wait()

        @pl.when(jnp.logical_not(pots[j]))
        def _():
            out_copy(j).start()

    def p1_last(sub):
        in_copy(NSUB - 1, sub).wait()

        @pl.when(jnp.logical_not(pots[NSUB - 1]))
        def _():
            out_copy(NSUB - 1, sub).start()

    branch_last(p1_last)

    plsc.subcore_barrier()
    pltpu.sync_copy(sh_hits, allhits_v)
    hit = []
    for k in range(NKV):
        h = allhits_v[0, pl.ds(k * L, L)]
        for r in range(1, NS):
            h = jnp.maximum(h, allhits_v[r, pl.ds(k * L, L)])
        hit.append(h)

    vval = val_v[...]

    def patch(j):
        sb, se = sub_bounds(j)
        for k in range(NKV):
            m = (hit[k] > 0) & (ts[k] >= sb) & (ts[k] < se)
            local = jnp.where(m, ts[k] - base, 0)
            plsc.store_scatter(vbuf, [local], vval, mask=m)

    for j in range(NSUB - 1):
        @pl.when(pots[j])
        def _():
            patch(j)
            out_copy(j).start()

    def p2_last(sub):
        @pl.when(pots[NSUB - 1])
        def _():
            patch(NSUB - 1)
            out_copy(NSUB - 1, sub).start()

    branch_last(p2_last)

    for j in range(NSUB - 1):
        out_copy(j).wait()
    branch_last(lambda sub: out_copy(NSUB - 1, sub).wait())


def kernel(el_ids, property_values, elements_in_region, value):
    ids = el_ids.astype(jnp.int32)
    eir_p = jnp.pad(elements_in_region.astype(jnp.int32),
                    (0, NUM_REGIONS_PAD - elements_in_region.shape[0]),
                    constant_values=-1)
    val16 = jnp.broadcast_to(jnp.asarray(value, jnp.float32), (L,))
    return _region_set_kernel(ids, eir_p, val16, property_values)


# speedup vs baseline: 1.0181x; 1.0181x over previous
import functools

import jax
import jax.numpy as jnp
from jax import lax
from jax.experimental import pallas as pl
from jax.experimental.pallas import tpu as pltpu, tpu_sc as plsc

NELEM = 1_000_000
NUM_IDS = 16_384
NUM_REGIONS_PAD = 64
NC, NS, L = 2, 16, 16
NW = NC * NS
NSUB = 4
SUB = 7_816
CHUNK = NSUB * SUB
SUB_LAST = NELEM - 31 * CHUNK - 3 * SUB
IDS_PER_TILE = NUM_IDS // NS
NKV = NUM_REGIONS_PAD // L

_MESH = plsc.VectorSubcoreMesh(core_axis_name="c", subcore_axis_name="s")


@functools.partial(
    pl.kernel,
    out_type=jax.ShapeDtypeStruct((NELEM,), jnp.float32),
    mesh=_MESH,
    scratch_types=[
        pltpu.VMEM((CHUNK,), jnp.float32),
        pltpu.VMEM((IDS_PER_TILE,), jnp.int32),
        pltpu.VMEM((NUM_REGIONS_PAD,), jnp.int32),
        pltpu.VMEM((NS, NUM_REGIONS_PAD), jnp.int32),
        pltpu.VMEM((NUM_REGIONS_PAD,), jnp.int32),
        pltpu.VMEM((L,), jnp.float32),
        pltpu.VMEM_SHARED((NS, NUM_REGIONS_PAD), jnp.int32),
        pltpu.SemaphoreType.DMA,
        pltpu.SemaphoreType.DMA,
        pltpu.SemaphoreType.DMA,
    ],
    compiler_params=pltpu.CompilerParams(needs_layout_passes=False),
)
def _region_set_kernel(ids_hbm, eir_hbm, val_hbm, prop_hbm, out_hbm,
                       vbuf, ids_v, hits_v, allhits_v, eir_v, val_v,
                       sh_hits, in_sem, out_sem, cfg_sem):
    c = lax.axis_index("c")
    s = lax.axis_index("s")
    wid = s * NC + c
    base = pl.multiple_of(wid * CHUNK, 8)
    last = wid == NW - 1

    def in_copy(j, sub=SUB):
        return pltpu.make_async_copy(
            prop_hbm.at[pl.ds(base + j * SUB, sub)],
            vbuf.at[pl.ds(j * SUB, sub)], in_sem)

    def out_copy(j, sub=SUB):
        return pltpu.make_async_copy(
            vbuf.at[pl.ds(j * SUB, sub)],
            out_hbm.at[pl.ds(base + j * SUB, sub)], out_sem)

    def branch_last(fn):
        @pl.when(jnp.logical_not(last))
        def _():
            fn(SUB)

        @pl.when(last)
        def _():
            fn(SUB_LAST)

    ids_cp = pltpu.make_async_copy(
        ids_hbm.at[pl.ds(s * IDS_PER_TILE, IDS_PER_TILE)], ids_v, cfg_sem)
    eir_cp = pltpu.make_async_copy(eir_hbm, eir_v, cfg_sem)
    val_cp = pltpu.make_async_copy(val_hbm, val_v, cfg_sem)
    ids_cp.start()
    eir_cp.start()
    val_cp.start()

    for j in range(NSUB - 1):
        in_copy(j).start()
    branch_last(lambda sub: in_copy(NSUB - 1, sub).start())

    ids_cp.wait()
    eir_cp.wait()
    val_cp.wait()

    ts = [eir_v[pl.ds(k * L, L)] for k in range(NKV)]
    chunk_end = base + jnp.where(last, 3 * SUB + SUB_LAST, CHUNK)

    def sub_bounds(j):
        sb = base + j * SUB
        se = jnp.minimum(sb + SUB, chunk_end)
        return sb, se

    zero16 = jnp.zeros((L,), jnp.int32)
    one16 = jnp.full((L,), 1, jnp.int32)
    for k in range(NKV):
        hits_v[pl.ds(k * L, L)] = zero16
    for k in range(IDS_PER_TILE // L):
        idv = ids_v[pl.ds(k * L, L)]
        plsc.store_scatter(hits_v, [idv], one16)
    pltpu.sync_copy(hits_v, sh_hits.at[s])

    plsc.subcore_barrier()
    pltpu.sync_copy(sh_hits, allhits_v)
    hit = []
    for k in range(NKV):
        h = allhits_v[0, pl.ds(k * L, L)]
        for r in range(1, NS):
            h = jnp.maximum(h, allhits_v[r, pl.ds(k * L, L)])
        hit.append(h)

    vval = val_v[...]

    def patch(j):
        sb, se = sub_bounds(j)
        for k in range(NKV):
            m = (hit[k] > 0) & (ts[k] >= sb) & (ts[k] < se)
            local = jnp.where(m, ts[k] - base, 0)
            plsc.store_scatter(vbuf, [local], vval, mask=m)

    for j in range(NSUB - 1):
        in_copy(j).wait()
        patch(j)
        out_copy(j).start()

    def drain_last(sub):
        in_copy(NSUB - 1, sub).wait()
        patch(NSUB - 1)
        out_copy(NSUB - 1, sub).start()

    branch_last(drain_last)

    for j in range(NSUB - 1):
        out_copy(j).wait()
    branch_last(lambda sub: out_copy(NSUB - 1, sub).wait())


def kernel(el_ids, property_values, elements_in_region, value):
    ids = el_ids.astype(jnp.int32)
    eir_p = jnp.pad(elements_in_region.astype(jnp.int32),
                    (0, NUM_REGIONS_PAD - elements_in_region.shape[0]),
                    constant_values=-1)
    val16 = jnp.broadcast_to(jnp.asarray(value, jnp.float32), (L,))
    return _region_set_kernel(ids, eir_p, val16, property_values)
